# baseline (speedup 1.0000x reference)
import functools

import jax
import jax.numpy as jnp
from jax import lax
from jax.experimental import pallas as pl
from jax.experimental.pallas import tpu as pltpu
from jax.experimental.pallas import tpu_sc as plsc

N = 10000
E = 160000
FN = 78
H = 64
W2 = 128
OD = 128
B = 64
PD = 1000

Ep = 163840
NCORES = 2
NSUB = 16
NW = NCORES * NSUB
EPT = Ep // NW
EPC = Ep // NSUB
CH = 128
NP = 10240
NROWS_T = NP // NSUB

_mesh = plsc.VectorSubcoreMesh(core_axis_name="c", subcore_axis_name="s")


@functools.partial(
    pl.kernel, mesh=_mesh,
    out_type=jax.ShapeDtypeStruct((NCORES, NP, W2), jnp.float32),
    scratch_types=[
        pltpu.VMEM_SHARED((NP, W2), jnp.float32),
        pltpu.VMEM((CH, W2), jnp.float32),
        pltpu.VMEM((CH,), jnp.int32),
    ])
def _sc_scatter(vals_hbm, row_hbm, col_hbm, z_hbm, out_hbm, tab, vbuf, ibuf):
    c = lax.axis_index("c")
    s = lax.axis_index("s")
    pltpu.sync_copy(z_hbm, vbuf)

    def zs(j, carry):
        pltpu.sync_copy(vbuf, tab.at[pl.ds(s * NROWS_T + j * CH, CH)])
        return carry

    lax.fori_loop(0, NROWS_T // CH, zs, 0)
    plsc.subcore_barrier()

    def make_chunk(idx_hbm):
        def chunk(i, carry):
            off = s * EPC + i * CH
            pltpu.sync_copy(vals_hbm.at[pl.ds(off, CH)], vbuf)
            pltpu.sync_copy(idx_hbm.at[pl.ds(off, CH)], ibuf)
            pltpu.sync_copy(vbuf, tab.at[ibuf], add=True)
            return carry
        return chunk

    @pl.when(c == 0)
    def _():
        lax.fori_loop(0, EPC // CH, make_chunk(row_hbm), 0)

    @pl.when(c == 1)
    def _():
        lax.fori_loop(0, EPC // CH, make_chunk(col_hbm), 0)

    plsc.subcore_barrier()
    pltpu.sync_copy(tab.at[pl.ds(s * NROWS_T, NROWS_T)],
                    out_hbm.at[c, pl.ds(s * NROWS_T, NROWS_T)])


@functools.partial(
    pl.kernel, mesh=_mesh,
    out_type=jax.ShapeDtypeStruct((Ep, W2), jnp.float32),
    scratch_types=[
        pltpu.VMEM((CH,), jnp.int32),
        pltpu.VMEM((CH,), jnp.int32),
        pltpu.VMEM((CH,), jnp.int32),
        pltpu.VMEM((CH, W2), jnp.float32),
        pltpu.VMEM((CH, W2), jnp.float32),
        pltpu.VMEM((CH, W2), jnp.float32),
        pltpu.VMEM((CH, W2), jnp.float32),
        pltpu.SemaphoreType.DMA,
        pltpu.SemaphoreType.DMA,
        pltpu.SemaphoreType.DMA,
    ])
def _sc_gather(row_hbm, col_hbm, rev_hbm, u_hbm, s_hbm, e_hbm, out_hbm,
               ir, ic, iv, ubuf, sbuf, ebuf, obuf, s1, s2, s3):
    c = lax.axis_index("c")
    s = lax.axis_index("s")
    wid = s * NCORES + c

    def chunk(i, carry):
        off = wid * EPT + i * CH
        pltpu.sync_copy(row_hbm.at[pl.ds(off, CH)], ir)
        pltpu.sync_copy(col_hbm.at[pl.ds(off, CH)], ic)
        pltpu.sync_copy(rev_hbm.at[pl.ds(off, CH)], iv)
        cp1 = pltpu.async_copy(u_hbm.at[ir], ubuf, s1)
        cp2 = pltpu.async_copy(s_hbm.at[ic], sbuf, s2)
        cp3 = pltpu.async_copy(e_hbm.at[iv], ebuf, s3)
        cp1.wait()
        cp2.wait()
        cp3.wait()

        def rowfn(r, acc):
            for q in range(H // 16):
                sl = pl.ds(q * 16, 16)
                sr = pl.ds(H + q * 16, 16)
                obuf[r, sl] = sbuf[r, sl] - ebuf[r, sl]
                obuf[r, sr] = ubuf[r, sl]
            return acc

        lax.fori_loop(0, CH, rowfn, 0)
        pltpu.sync_copy(obuf, out_hbm.at[pl.ds(off, CH)])
        return carry

    lax.fori_loop(0, EPT // CH, chunk, 0)


BLK_E = 2048
NB_E = Ep // BLK_E
BLK_N = 2000
NB_N = N // BLK_N


def _dot_t(a, w, prec=lax.Precision.HIGHEST):
    return lax.dot_general(a, w, (((1,), (1,)), ((), ())),
                           precision=prec,
                           preferred_element_type=jnp.float32)


def _dot_td(a, w):
    return lax.dot_general(a, w, (((1,), (1,)), ((), ())),
                           preferred_element_type=jnp.float32)


def _edge_init_body(ea_ref, we_ref, be_ref, eh_ref):
    i = pl.program_id(0)
    eh = _dot_t(ea_ref[...], we_ref[...]) + be_ref[...]
    rid = i * BLK_E + lax.broadcasted_iota(jnp.int32, (BLK_E, H), 0)
    eh = jnp.where(rid < E, eh, 0.0)
    eh_ref[...] = jnp.concatenate([eh, jnp.zeros_like(eh)], axis=1)


def _tc_edge_init(ea_p, W_edge_p, b_edge_r):
    return pl.pallas_call(
        _edge_init_body,
        grid=(NB_E,),
        in_specs=[
            pl.BlockSpec((BLK_E, 8), lambda i: (i, 0)),
            pl.BlockSpec((H, 8), lambda i: (0, 0)),
            pl.BlockSpec((1, H), lambda i: (0, 0)),
        ],
        out_specs=pl.BlockSpec((BLK_E, W2), lambda i: (i, 0)),
        out_shape=jax.ShapeDtypeStruct((Ep, W2), jnp.float32),
    )(ea_p, W_edge_p, b_edge_r)


def _edge_body(g_ref, bm_ref, wmb_ref, eh_ref):
    i = pl.program_id(0)
    g = g_ref[...]
    pre = _dot_t(g[:, :H], wmb_ref[...]) + g[:, H:] + bm_ref[...]
    m = jnp.maximum(pre, 0.0)
    rid = i * BLK_E + lax.broadcasted_iota(jnp.int32, (BLK_E, H), 0)
    m = jnp.where(rid < E, m, 0.0)
    eh_ref[...] = jnp.concatenate([m, jnp.zeros_like(m)], axis=1)


def _tc_edge(gath, bm, WmB):
    return pl.pallas_call(
        _edge_body,
        grid=(NB_E,),
        in_specs=[
            pl.BlockSpec((BLK_E, W2), lambda i: (i, 0)),
            pl.BlockSpec((1, H), lambda i: (0, 0)),
            pl.BlockSpec((H, H), lambda i: (0, 0)),
        ],
        out_specs=pl.BlockSpec((BLK_E, W2), lambda i: (i, 0)),
        out_shape=jax.ShapeDtypeStruct((Ep, W2), jnp.float32),
    )(gath, bm, WmB)


def _small1_body(x_ref, a1_ref, u_ref):
    u = _dot_t(x_ref[...], a1_ref[...])
    u_ref[...] = jnp.concatenate([u, jnp.zeros_like(u)], axis=1)


def _tc_small1(x, A1):
    return pl.pallas_call(
        _small1_body,
        out_shape=jax.ShapeDtypeStruct((N, W2), jnp.float32),
    )(x, A1)


def _small2_body(ncf_ref, xprev_ref, wna_ref, wnb_ref, bn_ref, a_ref,
                 x_ref, u_ref):
    nc = ncf_ref[:, :H]
    xl = jnp.maximum(_dot_t(xprev_ref[...], wna_ref[...])
                     + _dot_t(nc, wnb_ref[...]) + bn_ref[...], 0.0)
    x_ref[...] = xl
    u = _dot_t(xl, a_ref[...])
    u_ref[...] = jnp.concatenate([u, jnp.zeros_like(u)], axis=1)


def _tc_small2(NCf, x_prev, WnA, WnB, bn, A):
    return pl.pallas_call(
        _small2_body,
        grid=(1,),
        in_specs=[
            pl.BlockSpec((N, W2), lambda i: (0, 0)),
            pl.BlockSpec(x_prev.shape, lambda i: (0, 0)),
            pl.BlockSpec(WnA.shape, lambda i: (0, 0)),
            pl.BlockSpec(WnB.shape, lambda i: (0, 0)),
            pl.BlockSpec(bn.shape, lambda i: (0, 0)),
            pl.BlockSpec(A.shape, lambda i: (0, 0)),
        ],
        out_specs=[
            pl.BlockSpec((N, H), lambda i: (0, 0)),
            pl.BlockSpec((N, W2), lambda i: (0, 0)),
        ],
        out_shape=[
            jax.ShapeDtypeStruct((N, H), jnp.float32),
            jax.ShapeDtypeStruct((N, W2), jnp.float32),
        ],
    )(NCf, x_prev, WnA, WnB, bn, A)


def _final_body(ncf_ref, xprev_ref, wna_ref, wnb_ref, bn_ref, batch_ref,
                xp_ref):
    i = pl.program_id(0)
    nc = ncf_ref[:, :H]
    x3 = jnp.maximum(_dot_t(xprev_ref[...], wna_ref[...])
                     + _dot_t(nc, wnb_ref[...]) + bn_ref[...], 0.0)
    bids = batch_ref[0, 0, :]
    onehot = (lax.broadcasted_iota(jnp.int32, (B, BLK_N), 0)
              == bids[None, :]).astype(jnp.float32)
    part = lax.dot_general(onehot, x3, (((1,), (0,)), ((), ())),
                           precision=lax.Precision.HIGHEST,
                           preferred_element_type=jnp.float32)

    @pl.when(i == 0)
    def _():
        xp_ref[...] = jnp.zeros_like(xp_ref)

    xp_ref[...] += part


def _tc_final(NCf, x_prev, WnA, WnB, bn, batch3):
    return pl.pallas_call(
        _final_body,
        grid=(NB_N,),
        in_specs=[
            pl.BlockSpec((BLK_N, W2), lambda i: (i, 0)),
            pl.BlockSpec((BLK_N, H), lambda i: (i, 0)),
            pl.BlockSpec((H, H), lambda i: (0, 0)),
            pl.BlockSpec((H, H), lambda i: (0, 0)),
            pl.BlockSpec((1, H), lambda i: (0, 0)),
            pl.BlockSpec((1, 1, BLK_N), lambda i: (i, 0, 0)),
        ],
        out_specs=pl.BlockSpec((B, H), lambda i: (0, 0)),
        out_shape=jax.ShapeDtypeStruct((B, H), jnp.float32),
    )(NCf, x_prev, WnA, WnB, bn, batch3)


def _head_body(xp_ref, tg_ref, wxd_ref, bxd_ref, wp_ref, bp_ref,
               w1_ref, b1_ref, w2_ref, b2_ref, wo_ref, bo_ref, out_ref):
    xd = jnp.maximum(_dot_td(xp_ref[...], wxd_ref[...]) + bxd_ref[...], 0.0)
    xt = jnp.maximum(_dot_td(tg_ref[...], wp_ref[...]) + bp_ref[...], 0.0)
    h1 = jnp.maximum(_dot_td(xd, w1_ref[:, :OD]) + _dot_td(xt, w1_ref[:, OD:])
                     + b1_ref[...], 0.0)
    h2 = jnp.maximum(_dot_td(h1, w2_ref[...]) + b2_ref[...], 0.0)
    out_ref[...] = _dot_td(h2, wo_ref[...]) + bo_ref[...]


def _tc_head(xp, target, W_fcxd, b_fcxd, W_prot, b_prot,
             W_fc1, b_fc1, W_fc2, b_fc2, W_out, b_out):
    return pl.pallas_call(
        _head_body,
        out_shape=jax.ShapeDtypeStruct((B, 8), jnp.float32),
    )(xp, target, W_fcxd, b_fcxd, W_prot, b_prot,
      W_fc1, b_fc1, W_fc2, b_fc2, W_out, b_out)


def kernel(x, edge_attr, target, W_edge, b_edge, Wm1, bm1, Wn1, bn1,
           Wm2, bm2, Wn2, bn2, Wm3, bm3, Wn3, bn3, W_fcxd, b_fcxd,
           W_prot, b_prot, W_fc1, b_fc1, W_fc2, b_fc2, W_out, b_out,
           edge_index, batch):
    row, col = edge_index[0], edge_index[1]
    k = row * N + col
    rk = col * N + row
    order = jnp.argsort(k)
    sk = k[order]
    pos = jnp.clip(jnp.searchsorted(sk, rk), 0, E - 1)
    found = sk[pos] == rk
    rev = jnp.where(found, order[pos], E).astype(jnp.int32)

    PADN = Ep - E
    row_p = jnp.pad(row, (0, PADN))
    col_p = jnp.pad(col, (0, PADN))
    rev_p = jnp.pad(rev, (0, PADN), constant_values=E)
    ea_p = jnp.pad(edge_attr, ((0, PADN), (0, 2)))
    zrows = jnp.broadcast_to(x[0, 0] * 0.0, (CH, W2))
    batch3 = batch.reshape(NB_N, 1, BLK_N)

    A1, B1 = Wm1[:, :FN], Wm1[:, FN:]
    A2, B2 = Wm2[:, :H], Wm2[:, H:]
    A3, B3 = Wm3[:, :H], Wm3[:, H:]
    WnA1, WnB1 = Wn1[:, :FN], Wn1[:, FN:]
    WnA2, WnB2 = Wn2[:, :H], Wn2[:, H:]
    WnA3, WnB3 = Wn3[:, :H], Wn3[:, H:]
    bm1r, bm2r, bm3r = bm1.reshape(1, H), bm2.reshape(1, H), bm3.reshape(1, H)
    bn1r, bn2r, bn3r = bn1.reshape(1, H), bn2.reshape(1, H), bn3.reshape(1, H)
    W_edge_p = jnp.pad(W_edge, ((0, 0), (0, 2)))

    EH1 = _tc_edge_init(ea_p, W_edge_p, b_edge.reshape(1, H))
    S1 = _sc_scatter(EH1, row_p, col_p, zrows)
    U1 = _tc_small1(x, A1)
    G1 = _sc_gather(row_p, col_p, rev_p, U1, S1[0], EH1)
    EH2 = _tc_edge(G1, bm1r, B1)

    S2 = _sc_scatter(EH2, row_p, col_p, zrows)
    x1, U2 = _tc_small2(S2[1], x, WnA1, WnB1, bn1r, A2)
    G2 = _sc_gather(row_p, col_p, rev_p, U2, S2[0], EH2)
    EH3 = _tc_edge(G2, bm2r, B2)

    S3 = _sc_scatter(EH3, row_p, col_p, zrows)
    x2, U3 = _tc_small2(S3[1], x1, WnA2, WnB2, bn2r, A3)
    G3 = _sc_gather(row_p, col_p, rev_p, U3, S3[0], EH3)
    EH4 = _tc_edge(G3, bm3r, B3)

    S4 = _sc_scatter(EH4, row_p, col_p, zrows)
    xp = _tc_final(S4[1], x2, WnA3, WnB3, bn3r, batch3)

    W_out_p = jnp.pad(W_out, ((0, 7), (0, 0)))
    b_out_p = jnp.pad(b_out.reshape(1, 1), ((0, 0), (0, 7)))
    res = _tc_head(xp, target, W_fcxd, b_fcxd.reshape(1, OD),
                   W_prot, b_prot.reshape(1, OD),
                   W_fc1, b_fc1.reshape(1, 1024),
                   W_fc2, b_fc2.reshape(1, 2 * OD),
                   W_out_p, b_out_p)
    return res[:, :1]
